# baseline (speedup 1.0000x reference)
import functools

import jax
import jax.numpy as jnp
from jax import lax
from jax.experimental import pallas as pl
from jax.experimental.pallas import tpu as pltpu
from jax.experimental.pallas import tpu_sc as plsc

_PERIOD = 840
_D = 64
_WINDOW = 256


def _prep_body(ts_ref, w_ref, m_ref, s_ref, idx_ref, c_ref):
    idx_ref[...] = lax.rem(ts_ref[...], _PERIOD)

    def onehot(n):
        r = lax.broadcasted_iota(jnp.int32, (_PERIOD, n), 0)
        c = lax.broadcasted_iota(jnp.int32, (_PERIOD, n), 1)
        return (lax.rem(r, n) == c).astype(jnp.float32)

    dot = functools.partial(
        jnp.dot,
        preferred_element_type=jnp.float32,
        precision=lax.Precision.HIGHEST,
    )
    c_ref[...] = (
        dot(onehot(7), w_ref[...])
        + dot(onehot(30), m_ref[...])
        + dot(onehot(120), s_ref[...])
    )


def _prep(ts, week_embed, month_embed, season_embed):
    return pl.pallas_call(
        _prep_body,
        out_shape=(
            jax.ShapeDtypeStruct(ts.shape, jnp.int32),
            jax.ShapeDtypeStruct((_PERIOD, _D), jnp.float32),
        ),
    )(ts, week_embed, month_embed, season_embed)


def _sc_gather(table, idx2d, n):
    mesh = plsc.VectorSubcoreMesh(
        core_axis_name="core", subcore_axis_name="subcore"
    )

    @functools.partial(
        pl.kernel,
        out_type=jax.ShapeDtypeStruct((n, _D), jnp.float32),
        mesh=mesh,
        compiler_params=pltpu.CompilerParams(use_tc_tiling_on_sc=False),
    )
    def gather_kernel(c_hbm, i_hbm, o_hbm):
        def body(i_vmem, o_vmem):
            pltpu.sync_copy(c_hbm.at[i_vmem.at[0]], o_vmem)

        pltpu.emit_pipeline(
            body,
            grid=(n // _WINDOW,),
            in_specs=[
                pl.BlockSpec((1, _WINDOW), index_map=lambda i: (0, i))
            ],
            out_specs=[
                pl.BlockSpec((_WINDOW, _D), index_map=lambda i: (i, 0))
            ],
            core_axis_name=("core", "subcore"),
            dimension_semantics=(pltpu.PARALLEL,),
        )(i_hbm, o_hbm)

    return gather_kernel(table, idx2d)


def _transpose_body(in_ref, out_ref):
    x3 = in_ref[...].reshape(128, 100, 128)
    y = jnp.transpose(x3, (1, 0, 2))
    y = jnp.transpose(y, (0, 2, 1))
    out_ref[...] = y.reshape(12800, 128)


def _tc_transpose(flat2d, b, t):
    rows = t * _D
    n_blocks = b // 128
    return pl.pallas_call(
        _transpose_body,
        grid=(n_blocks,),
        in_specs=[
            pl.BlockSpec((rows, 128), lambda i: (i, 0)),
        ],
        out_specs=pl.BlockSpec((rows, 128), lambda i: (0, i)),
        out_shape=jax.ShapeDtypeStruct((rows, b), jnp.float32),
    )(flat2d)


def kernel(ts, week_embed, month_embed, season_embed):
    b, t = ts.shape
    n = b * t
    idx, table = _prep(ts, week_embed, month_embed, season_embed)
    out = _sc_gather(table, idx.reshape(1, n), n)
    flat2d = out.reshape(n * _D // 128, 128)
    out_t = _tc_transpose(flat2d, b, t)
    return out_t.reshape(t, _D, b).transpose(2, 0, 1)


# speedup vs baseline: 2.1867x; 2.1867x over previous
import functools

import jax
import jax.numpy as jnp
from jax import lax
from jax.experimental import pallas as pl
from jax.experimental.pallas import tpu as pltpu
from jax.experimental.pallas import tpu_sc as plsc

_PERIOD = 840
_D = 64
_WINDOW = 64


def _prep_body(ts_ref, w_ref, m_ref, s_ref, idx_ref, c_ref):
    idx_ref[...] = lax.rem(ts_ref[...], _PERIOD)

    def onehot(n):
        r = lax.broadcasted_iota(jnp.int32, (_PERIOD, n), 0)
        c = lax.broadcasted_iota(jnp.int32, (_PERIOD, n), 1)
        return (lax.rem(r, n) == c).astype(jnp.float32)

    dot = functools.partial(
        jnp.dot,
        preferred_element_type=jnp.float32,
        precision=lax.Precision.HIGHEST,
    )
    c_ref[...] = (
        dot(onehot(7), w_ref[...])
        + dot(onehot(30), m_ref[...])
        + dot(onehot(120), s_ref[...])
    )


def _prep(ts, week_embed, month_embed, season_embed):
    return pl.pallas_call(
        _prep_body,
        out_shape=(
            jax.ShapeDtypeStruct(ts.shape, jnp.int32),
            jax.ShapeDtypeStruct((_PERIOD, _D), jnp.float32),
        ),
    )(ts, week_embed, month_embed, season_embed)


def _sc_gather(table, idx2d, n):
    mesh = plsc.VectorSubcoreMesh(
        core_axis_name="core", subcore_axis_name="subcore"
    )

    @functools.partial(
        pl.kernel,
        out_type=jax.ShapeDtypeStruct((n, _D), jnp.float32),
        mesh=mesh,
        compiler_params=pltpu.CompilerParams(use_tc_tiling_on_sc=False),
    )
    def gather_kernel(c_hbm, i_hbm, o_hbm):
        def body(i_vmem, o_vmem):
            pltpu.sync_copy(c_hbm.at[i_vmem.at[0]], o_vmem)

        pltpu.emit_pipeline(
            body,
            grid=(n // _WINDOW,),
            in_specs=[
                pl.BlockSpec((1, _WINDOW), index_map=lambda i: (0, i))
            ],
            out_specs=[
                pl.BlockSpec((_WINDOW, _D), index_map=lambda i: (i, 0))
            ],
            core_axis_name=("core", "subcore"),
            dimension_semantics=(pltpu.PARALLEL,),
        )(i_hbm, o_hbm)

    return gather_kernel(table, idx2d)


def _transpose_body(in_ref, out_ref):
    x3 = in_ref[...].reshape(128, 100, 128)
    y = jnp.transpose(x3, (1, 0, 2))
    y = jnp.transpose(y, (0, 2, 1))
    out_ref[...] = y.reshape(12800, 128)


def _tc_transpose(flat2d, b, t):
    rows = t * _D
    n_blocks = b // 128
    return pl.pallas_call(
        _transpose_body,
        grid=(n_blocks,),
        in_specs=[
            pl.BlockSpec((rows, 128), lambda i: (i, 0)),
        ],
        out_specs=pl.BlockSpec((rows, 128), lambda i: (0, i)),
        out_shape=jax.ShapeDtypeStruct((rows, b), jnp.float32),
    )(flat2d)


def kernel(ts, week_embed, month_embed, season_embed):
    b, t = ts.shape
    n = b * t
    idx, table = _prep(ts, week_embed, month_embed, season_embed)
    out = _sc_gather(table, idx.reshape(1, n), n)
    flat2d = out.reshape(n * _D // 128, 128)
    out_t = _tc_transpose(flat2d, b, t)
    return out_t.reshape(t, _D, b).transpose(2, 0, 1)


# speedup vs baseline: 3.3178x; 1.5173x over previous
import functools

import jax
import jax.numpy as jnp
from jax import lax
from jax.experimental import pallas as pl
from jax.experimental.pallas import tpu as pltpu
from jax.experimental.pallas import tpu_sc as plsc

_PERIOD = 840
_D = 64
_WINDOW = 32


def _prep_body(ts_ref, w_ref, m_ref, s_ref, idx_ref, c_ref):
    idx_ref[...] = lax.rem(ts_ref[...], _PERIOD)

    def onehot(n):
        r = lax.broadcasted_iota(jnp.int32, (_PERIOD, n), 0)
        c = lax.broadcasted_iota(jnp.int32, (_PERIOD, n), 1)
        return (lax.rem(r, n) == c).astype(jnp.float32)

    dot = functools.partial(
        jnp.dot,
        preferred_element_type=jnp.float32,
        precision=lax.Precision.HIGHEST,
    )
    c_ref[...] = (
        dot(onehot(7), w_ref[...])
        + dot(onehot(30), m_ref[...])
        + dot(onehot(120), s_ref[...])
    )


def _prep(ts, week_embed, month_embed, season_embed):
    return pl.pallas_call(
        _prep_body,
        out_shape=(
            jax.ShapeDtypeStruct(ts.shape, jnp.int32),
            jax.ShapeDtypeStruct((_PERIOD, _D), jnp.float32),
        ),
    )(ts, week_embed, month_embed, season_embed)


def _sc_gather(table, idx2d, n):
    mesh = plsc.VectorSubcoreMesh(
        core_axis_name="core", subcore_axis_name="subcore"
    )

    @functools.partial(
        pl.kernel,
        out_type=jax.ShapeDtypeStruct((n, _D), jnp.float32),
        mesh=mesh,
        compiler_params=pltpu.CompilerParams(use_tc_tiling_on_sc=False),
    )
    def gather_kernel(c_hbm, i_hbm, o_hbm):
        def body(i_vmem, o_vmem):
            pltpu.sync_copy(c_hbm.at[i_vmem.at[0]], o_vmem)

        pltpu.emit_pipeline(
            body,
            grid=(n // _WINDOW,),
            in_specs=[
                pl.BlockSpec((1, _WINDOW), index_map=lambda i: (0, i))
            ],
            out_specs=[
                pl.BlockSpec((_WINDOW, _D), index_map=lambda i: (i, 0))
            ],
            core_axis_name=("core", "subcore"),
            dimension_semantics=(pltpu.PARALLEL,),
        )(i_hbm, o_hbm)

    return gather_kernel(table, idx2d)


def _transpose_body(in_ref, out_ref):
    x3 = in_ref[...].reshape(128, 100, 128)
    y = jnp.transpose(x3, (1, 0, 2))
    y = jnp.transpose(y, (0, 2, 1))
    out_ref[...] = y.reshape(12800, 128)


def _tc_transpose(flat2d, b, t):
    rows = t * _D
    n_blocks = b // 128
    return pl.pallas_call(
        _transpose_body,
        grid=(n_blocks,),
        in_specs=[
            pl.BlockSpec((rows, 128), lambda i: (i, 0)),
        ],
        out_specs=pl.BlockSpec((rows, 128), lambda i: (0, i)),
        out_shape=jax.ShapeDtypeStruct((rows, b), jnp.float32),
    )(flat2d)


def kernel(ts, week_embed, month_embed, season_embed):
    b, t = ts.shape
    n = b * t
    idx, table = _prep(ts, week_embed, month_embed, season_embed)
    out = _sc_gather(table, idx.reshape(1, n), n)
    flat2d = out.reshape(n * _D // 128, 128)
    out_t = _tc_transpose(flat2d, b, t)
    return out_t.reshape(t, _D, b).transpose(2, 0, 1)
